# baseline (speedup 1.0000x reference)
import jax
import jax.numpy as jnp
from jax.experimental import pallas as pl


def _posemb_kernel(be_ref, bo_ref, tabp_ref, out_ref):
    bt, hp, w = out_ref.shape
    e = w // 2
    me = be_ref[...] != 0
    mo = bo_ref[...] != 0
    me3 = jnp.swapaxes(jax.lax.broadcast_in_dim(me, (bt, 1, hp), (0, 2)), 1, 2)
    mo3 = jnp.swapaxes(jax.lax.broadcast_in_dim(mo, (bt, 1, hp), (0, 2)), 1, 2)
    tabp = tabp_ref[...]
    left = jnp.where(me3, tabp, 0.0)
    right = jnp.where(mo3, tabp, 0.0)
    lane = jax.lax.broadcasted_iota(jnp.int32, (bt, hp, w), 2)
    out_ref[...] = jnp.where(lane < e, left, right)


def kernel(batch, emb_table):
    B, L = batch.shape
    E = emb_table.shape[1]
    HP = L // 2
    W = 2 * E
    tabp = emb_table[1:L + 1].reshape(1, HP, W)
    be = batch[:, 0::2]
    bo = batch[:, 1::2]
    BT = 256
    grid = (B // BT,)
    out = pl.pallas_call(
        _posemb_kernel,
        grid=grid,
        in_specs=[
            pl.BlockSpec((BT, HP), lambda i: (i, 0)),
            pl.BlockSpec((BT, HP), lambda i: (i, 0)),
            pl.BlockSpec((1, HP, W), lambda i: (0, 0, 0)),
        ],
        out_specs=pl.BlockSpec((BT, HP, W), lambda i: (i, 0, 0)),
        out_shape=jax.ShapeDtypeStruct((B, HP, W), jnp.float32),
    )(be, bo, tabp)
    return out.reshape(B, L, E)
